# baseline (speedup 1.0000x reference)
import jax
import jax.numpy as jnp
from jax import lax
from jax.experimental import pallas as pl
from jax.experimental.pallas import tpu as pltpu
from jax.experimental.pallas import tpu_sc as plsc

N_ROWS = 320000
N_COLS = 128
N_SEG = 10000
N_SEG_PAD = 10240
CHUNK = 128
N_CHUNKS = N_ROWS // CHUNK
N_WORKERS = 32
STEPS = (N_CHUNKS + N_WORKERS - 1) // N_WORKERS
SEG_SLICE = N_SEG_PAD // 16


def _mesh():
    return plsc.VectorSubcoreMesh(core_axis_name="c", subcore_axis_name="s")


def _k0_body(ids_hbm, outc_hbm, cnts_sh, ids_v, ids_v1, ones_v, czb_v, sem0, sem1):
    c = lax.axis_index("c")
    s = lax.axis_index("s")
    wid = s * 2 + c

    def init_ones(i, _):
        for j in range(8):
            ones_v[i, pl.ds(j * 16, 16)] = jnp.ones((16,), jnp.float32)
            czb_v[i, pl.ds(j * 16, 16)] = jnp.zeros((16,), jnp.float32)
        return 0
    lax.fori_loop(0, CHUNK, init_ones, 0)

    for jj in range(5):
        pltpu.sync_copy(czb_v, cnts_sh.at[pl.ds(s * SEG_SLICE + jj * CHUNK, CHUNK)])
    plsc.subcore_barrier()

    bufs = ((ids_v, sem0), (ids_v1, sem1))

    def issue(k, b):
        idv, sem = bufs[b]
        chunk = k * N_WORKERS + wid

        @pl.when(chunk < N_CHUNKS)
        def _():
            pltpu.async_copy(ids_hbm.at[pl.ds(chunk * CHUNK, CHUNK)], idv, sem)

    def process(k, b):
        idv, sem = bufs[b]
        chunk = k * N_WORKERS + wid

        @pl.when(chunk < N_CHUNKS)
        def _():
            pltpu.make_async_copy(ids_hbm.at[pl.ds(0, CHUNK)], idv, sem).wait()
            pltpu.sync_copy(ones_v, cnts_sh.at[idv], add=True)

    issue(0, 0)

    def pair(p, _):
        issue(2 * p + 1, 1)
        process(2 * p, 0)
        issue(2 * p + 2, 0)
        process(2 * p + 1, 1)
        return 0
    lax.fori_loop(0, 40, pair, 0)
    plsc.subcore_barrier()

    pltpu.sync_copy(cnts_sh.at[pl.ds(s * SEG_SLICE, SEG_SLICE)],
                    outc_hbm.at[c, pl.ds(s * SEG_SLICE, SEG_SLICE)])


def _k1_body(x_hbm, ids_hbm, outs_hbm, sums_sh,
             x_v0, x_v1, ids_v0, ids_v1, sem_x0, sem_x1):
    c = lax.axis_index("c")
    s = lax.axis_index("s")
    wid = s * 2 + c

    def zrow(i, _):
        for j in range(8):
            x_v0[i, pl.ds(j * 16, 16)] = jnp.zeros((16,), jnp.float32)
        return 0
    lax.fori_loop(0, CHUNK, zrow, 0)

    for jj in range(5):
        pltpu.sync_copy(x_v0, sums_sh.at[pl.ds(s * SEG_SLICE + jj * CHUNK, CHUNK)])
    plsc.subcore_barrier()

    bufs = ((x_v0, ids_v0, sem_x0), (x_v1, ids_v1, sem_x1))

    def issue(k, b):
        x_v, idv, sem = bufs[b]
        chunk = k * N_WORKERS + wid

        @pl.when(chunk < N_CHUNKS)
        def _():
            base = chunk * CHUNK
            pltpu.sync_copy(ids_hbm.at[pl.ds(base, CHUNK)], idv)
            pltpu.async_copy(x_hbm.at[pl.ds(base, CHUNK)], x_v, sem)

    def process(k, b):
        x_v, idv, sem = bufs[b]
        chunk = k * N_WORKERS + wid

        @pl.when(chunk < N_CHUNKS)
        def _():
            pltpu.make_async_copy(x_hbm.at[pl.ds(0, CHUNK)], x_v, sem).wait()
            pltpu.sync_copy(x_v, sums_sh.at[idv], add=True)

    issue(0, 0)

    def pair(p, _):
        issue(2 * p + 1, 1)
        process(2 * p, 0)
        issue(2 * p + 2, 0)
        process(2 * p + 1, 1)
        return 0
    lax.fori_loop(0, 40, pair, 0)
    plsc.subcore_barrier()

    pltpu.sync_copy(sums_sh.at[pl.ds(s * SEG_SLICE, SEG_SLICE)],
                    outs_hbm.at[c, pl.ds(s * SEG_SLICE, SEG_SLICE)])


def _k2_body(s_ref, c_ref, o_ref):
    tot = s_ref[0] + s_ref[1]
    cnt = c_ref[0, :, 0:1] + c_ref[1, :, 0:1]
    o_ref[...] = -(tot / jnp.maximum(cnt, 1.0))


def _k3_body(x_hbm, ids_hbm, nmean_hbm, out_hbm,
             x_v0, x_v1, ids_v0, ids_v1,
             sem_x0, sem_x1, sem_w0, sem_w1):
    c = lax.axis_index("c")
    s = lax.axis_index("s")
    wid = s * 2 + c

    bufs = ((x_v0, ids_v0, sem_x0, sem_w0),
            (x_v1, ids_v1, sem_x1, sem_w1))

    def issue(k, b):
        x_v, ids_v, sem_x, sem_w = bufs[b]
        chunk = k * N_WORKERS + wid

        @pl.when(jnp.logical_and(k >= 2, chunk - 2 * N_WORKERS < N_CHUNKS))
        def _():
            pltpu.make_async_copy(x_v, out_hbm.at[pl.ds(0, CHUNK)], sem_w).wait()

        @pl.when(chunk < N_CHUNKS)
        def _():
            base = chunk * CHUNK
            pltpu.sync_copy(ids_hbm.at[pl.ds(base, CHUNK)], ids_v)
            pltpu.async_copy(x_hbm.at[pl.ds(base, CHUNK)], x_v, sem_x)

    def process(k, b):
        x_v, ids_v, sem_x, sem_w = bufs[b]
        chunk = k * N_WORKERS + wid

        @pl.when(chunk < N_CHUNKS)
        def _():
            base = chunk * CHUNK
            pltpu.make_async_copy(x_hbm.at[pl.ds(0, CHUNK)], x_v, sem_x).wait()
            pltpu.async_copy(nmean_hbm.at[ids_v], x_v, sem_x, add=True).wait()
            pltpu.async_copy(x_v, out_hbm.at[pl.ds(base, CHUNK)], sem_w)

    issue(0, 0)

    def pair(p, _):
        issue(2 * p + 1, 1)
        process(2 * p, 0)
        issue(2 * p + 2, 0)
        process(2 * p + 1, 1)
        return 0
    lax.fori_loop(0, 40, pair, 0)

    @pl.when(79 * N_WORKERS + wid < N_CHUNKS)
    def _():
        pltpu.make_async_copy(x_v1, out_hbm.at[pl.ds(0, CHUNK)], sem_w1).wait()


def kernel(x, batch):
    ids = batch.astype(jnp.int32)

    k0 = pl.kernel(
        _k0_body,
        out_type=jax.ShapeDtypeStruct((2, N_SEG_PAD, N_COLS), jnp.float32),
        mesh=_mesh(),
        scratch_types=[
            pltpu.VMEM_SHARED((N_SEG_PAD, N_COLS), jnp.float32),
            pltpu.VMEM((CHUNK,), jnp.int32),
            pltpu.VMEM((CHUNK,), jnp.int32),
            pltpu.VMEM((CHUNK, N_COLS), jnp.float32),
            pltpu.VMEM((CHUNK, N_COLS), jnp.float32),
            pltpu.SemaphoreType.DMA,
            pltpu.SemaphoreType.DMA,
        ],
    )
    part_c = k0(ids)

    k1 = pl.kernel(
        _k1_body,
        out_type=jax.ShapeDtypeStruct((2, N_SEG_PAD, N_COLS), jnp.float32),
        mesh=_mesh(),
        scratch_types=[
            pltpu.VMEM_SHARED((N_SEG_PAD, N_COLS), jnp.float32),
            pltpu.VMEM((CHUNK, N_COLS), jnp.float32),
            pltpu.VMEM((CHUNK, N_COLS), jnp.float32),
            pltpu.VMEM((CHUNK,), jnp.int32),
            pltpu.VMEM((CHUNK,), jnp.int32),
            pltpu.SemaphoreType.DMA,
            pltpu.SemaphoreType.DMA,
        ],
    )
    part_s = k1(x, ids)

    mean = pl.pallas_call(
        _k2_body,
        out_shape=jax.ShapeDtypeStruct((N_SEG_PAD, N_COLS), jnp.float32),
    )(part_s, part_c)

    k3 = pl.kernel(
        _k3_body,
        out_type=jax.ShapeDtypeStruct((N_ROWS, N_COLS), jnp.float32),
        mesh=_mesh(),
        scratch_types=[
            pltpu.VMEM((CHUNK, N_COLS), jnp.float32),
            pltpu.VMEM((CHUNK, N_COLS), jnp.float32),
            pltpu.VMEM((CHUNK,), jnp.int32),
            pltpu.VMEM((CHUNK,), jnp.int32),
            pltpu.SemaphoreType.DMA,
            pltpu.SemaphoreType.DMA,
            pltpu.SemaphoreType.DMA,
            pltpu.SemaphoreType.DMA,
        ],
    )
    return k3(x, ids, mean)


# speedup vs baseline: 1.1824x; 1.1824x over previous
import jax
import jax.numpy as jnp
from jax import lax
from jax.experimental import pallas as pl
from jax.experimental.pallas import tpu as pltpu
from jax.experimental.pallas import tpu_sc as plsc

N_ROWS = 320000
N_COLS = 128
N_SEG = 10000
N_SEG_PAD = 10240
CHUNK = 128
N_CHUNKS = N_ROWS // CHUNK
N_WORKERS = 32
STEPS = (N_CHUNKS + N_WORKERS - 1) // N_WORKERS
SEG_SLICE = N_SEG_PAD // 16


def _mesh():
    return plsc.VectorSubcoreMesh(core_axis_name="c", subcore_axis_name="s")


def _k0_body(ids_hbm, outc_hbm, cnts_sh, ids_v, ids_v1, ones_v, czb_v, sem0, sem1):
    c = lax.axis_index("c")
    s = lax.axis_index("s")
    wid = s * 2 + c

    def init_ones(i, _):
        for j in range(8):
            ones_v[i, pl.ds(j * 16, 16)] = jnp.ones((16,), jnp.float32)
            czb_v[i, pl.ds(j * 16, 16)] = jnp.zeros((16,), jnp.float32)
        return 0
    lax.fori_loop(0, CHUNK, init_ones, 0)

    for jj in range(5):
        pltpu.sync_copy(czb_v, cnts_sh.at[pl.ds(s * SEG_SLICE + jj * CHUNK, CHUNK)])
    plsc.subcore_barrier()

    bufs = ((ids_v, sem0), (ids_v1, sem1))

    def issue(k, b):
        idv, sem = bufs[b]
        chunk = k * N_WORKERS + wid

        @pl.when(chunk < N_CHUNKS)
        def _():
            pltpu.async_copy(ids_hbm.at[pl.ds(chunk * CHUNK, CHUNK)], idv, sem)

    def process(k, b):
        idv, sem = bufs[b]
        chunk = k * N_WORKERS + wid

        @pl.when(chunk < N_CHUNKS)
        def _():
            pltpu.make_async_copy(ids_hbm.at[pl.ds(0, CHUNK)], idv, sem).wait()
            pltpu.sync_copy(ones_v, cnts_sh.at[idv], add=True)

    issue(0, 0)

    def pair(p, _):
        issue(2 * p + 1, 1)
        process(2 * p, 0)
        issue(2 * p + 2, 0)
        process(2 * p + 1, 1)
        return 0
    lax.fori_loop(0, 40, pair, 0)
    plsc.subcore_barrier()

    pltpu.sync_copy(cnts_sh.at[pl.ds(s * SEG_SLICE, SEG_SLICE)],
                    outc_hbm.at[c, pl.ds(s * SEG_SLICE, SEG_SLICE)])


def _k1_body(x_hbm, ids_hbm, outs_hbm, sums_sh,
             x_v0, x_v1, ids_v0, ids_v1, sem_x0, sem_x1):
    c = lax.axis_index("c")
    s = lax.axis_index("s")
    wid = s * 2 + c

    def zrow(i, _):
        for j in range(8):
            x_v0[i, pl.ds(j * 16, 16)] = jnp.zeros((16,), jnp.float32)
        return 0
    lax.fori_loop(0, CHUNK, zrow, 0)

    for jj in range(5):
        pltpu.sync_copy(x_v0, sums_sh.at[pl.ds(s * SEG_SLICE + jj * CHUNK, CHUNK)])
    plsc.subcore_barrier()

    bufs = ((x_v0, ids_v0, sem_x0), (x_v1, ids_v1, sem_x1))

    def issue(k, b):
        x_v, idv, sem = bufs[b]
        chunk = k * N_WORKERS + wid

        @pl.when(chunk < N_CHUNKS)
        def _():
            base = chunk * CHUNK
            pltpu.sync_copy(ids_hbm.at[pl.ds(base, CHUNK)], idv)
            pltpu.async_copy(x_hbm.at[pl.ds(base, CHUNK)], x_v, sem)

    def process(k, b):
        x_v, idv, sem = bufs[b]
        chunk = k * N_WORKERS + wid

        @pl.when(chunk < N_CHUNKS)
        def _():
            pltpu.make_async_copy(x_hbm.at[pl.ds(0, CHUNK)], x_v, sem).wait()
            pltpu.sync_copy(x_v, sums_sh.at[idv], add=True)

    issue(0, 0)

    def pair(p, _):
        issue(2 * p + 1, 1)
        process(2 * p, 0)
        issue(2 * p + 2, 0)
        process(2 * p + 1, 1)
        return 0
    lax.fori_loop(0, 40, pair, 0)
    plsc.subcore_barrier()

    pltpu.sync_copy(sums_sh.at[pl.ds(s * SEG_SLICE, SEG_SLICE)],
                    outs_hbm.at[c, pl.ds(s * SEG_SLICE, SEG_SLICE)])


def _k2_body(s_ref, c_ref, o_ref):
    tot = s_ref[0] + s_ref[1]
    cnt = c_ref[0, :, 0:1] + c_ref[1, :, 0:1]
    o_ref[...] = -(tot / jnp.maximum(cnt, 1.0))


def _k3_body(x_hbm, ids_hbm, nmean_hbm, out_hbm,
             x_v0, x_v1, x_v2, ids_v0, ids_v1, ids_v2,
             sem_x0, sem_x1, sem_x2, sem_g0, sem_g1, sem_g2,
             sem_w0, sem_w1, sem_w2):
    c = lax.axis_index("c")
    s = lax.axis_index("s")
    wid = s * 2 + c

    X = (x_v0, x_v1, x_v2)
    I = (ids_v0, ids_v1, ids_v2)
    SX = (sem_x0, sem_x1, sem_x2)
    SG = (sem_g0, sem_g1, sem_g2)
    SW = (sem_w0, sem_w1, sem_w2)

    def stage(k, b):
        chunk = k * N_WORKERS + wid

        @pl.when(jnp.logical_and(k >= 3, chunk - 3 * N_WORKERS < N_CHUNKS))
        def _():
            pltpu.make_async_copy(X[b], out_hbm.at[pl.ds(0, CHUNK)], SW[b]).wait()

        @pl.when(chunk < N_CHUNKS)
        def _():
            base = chunk * CHUNK
            pltpu.sync_copy(ids_hbm.at[pl.ds(base, CHUNK)], I[b])
            pltpu.async_copy(x_hbm.at[pl.ds(base, CHUNK)], X[b], SX[b])

    def gadd(k, b):
        chunk = k * N_WORKERS + wid

        @pl.when(chunk < N_CHUNKS)
        def _():
            pltpu.make_async_copy(x_hbm.at[pl.ds(0, CHUNK)], X[b], SX[b]).wait()
            pltpu.async_copy(nmean_hbm.at[I[b]], X[b], SG[b], add=True)

    def write(k, b):
        chunk = k * N_WORKERS + wid

        @pl.when(chunk < N_CHUNKS)
        def _():
            pltpu.make_async_copy(x_hbm.at[pl.ds(0, CHUNK)], X[b], SG[b]).wait()
            pltpu.async_copy(X[b], out_hbm.at[pl.ds(chunk * CHUNK, CHUNK)], SW[b])

    stage(0, 0)
    stage(1, 1)
    gadd(0, 0)

    def tri(t, _):
        s0 = 3 * t + 2
        stage(s0, 2)
        gadd(s0 - 1, 1)
        write(s0 - 2, 0)
        stage(s0 + 1, 0)
        gadd(s0, 2)
        write(s0 - 1, 1)
        stage(s0 + 2, 1)
        gadd(s0 + 1, 0)
        write(s0, 2)
        return 0
    lax.fori_loop(0, 27, tri, 0)


def kernel(x, batch):
    ids = batch.astype(jnp.int32)

    k0 = pl.kernel(
        _k0_body,
        out_type=jax.ShapeDtypeStruct((2, N_SEG_PAD, N_COLS), jnp.float32),
        mesh=_mesh(),
        scratch_types=[
            pltpu.VMEM_SHARED((N_SEG_PAD, N_COLS), jnp.float32),
            pltpu.VMEM((CHUNK,), jnp.int32),
            pltpu.VMEM((CHUNK,), jnp.int32),
            pltpu.VMEM((CHUNK, N_COLS), jnp.float32),
            pltpu.VMEM((CHUNK, N_COLS), jnp.float32),
            pltpu.SemaphoreType.DMA,
            pltpu.SemaphoreType.DMA,
        ],
    )
    part_c = k0(ids)

    k1 = pl.kernel(
        _k1_body,
        out_type=jax.ShapeDtypeStruct((2, N_SEG_PAD, N_COLS), jnp.float32),
        mesh=_mesh(),
        scratch_types=[
            pltpu.VMEM_SHARED((N_SEG_PAD, N_COLS), jnp.float32),
            pltpu.VMEM((CHUNK, N_COLS), jnp.float32),
            pltpu.VMEM((CHUNK, N_COLS), jnp.float32),
            pltpu.VMEM((CHUNK,), jnp.int32),
            pltpu.VMEM((CHUNK,), jnp.int32),
            pltpu.SemaphoreType.DMA,
            pltpu.SemaphoreType.DMA,
        ],
    )
    part_s = k1(x, ids)

    mean = pl.pallas_call(
        _k2_body,
        out_shape=jax.ShapeDtypeStruct((N_SEG_PAD, N_COLS), jnp.float32),
    )(part_s, part_c)

    k3 = pl.kernel(
        _k3_body,
        out_type=jax.ShapeDtypeStruct((N_ROWS, N_COLS), jnp.float32),
        mesh=_mesh(),
        scratch_types=(
            [pltpu.VMEM((CHUNK, N_COLS), jnp.float32)] * 3
            + [pltpu.VMEM((CHUNK,), jnp.int32)] * 3
            + [pltpu.SemaphoreType.DMA] * 9
        ),
    )
    return k3(x, ids, mean)


# speedup vs baseline: 1.9708x; 1.6668x over previous
import jax
import jax.numpy as jnp
from jax import lax
from jax.experimental import pallas as pl
from jax.experimental.pallas import tpu as pltpu
from jax.experimental.pallas import tpu_sc as plsc

N_ROWS = 320000
N_COLS = 128
N_SEG = 10000
N_SEG_PAD = 10240
CHUNK = 128
N_CHUNKS = N_ROWS // CHUNK
N_WORKERS = 32
STEPS = (N_CHUNKS + N_WORKERS - 1) // N_WORKERS
SEG_SLICE = N_SEG_PAD // 16
N_SEG_K3 = 10112
SEG_SLICE_K3 = N_SEG_K3 // 16


def _mesh():
    return plsc.VectorSubcoreMesh(core_axis_name="c", subcore_axis_name="s")


def _k0_body(ids_hbm, outc_hbm, cnts_sh, ids_v, ids_v1, ones_v, czb_v, sem0, sem1):
    c = lax.axis_index("c")
    s = lax.axis_index("s")
    wid = s * 2 + c

    def init_ones(i, _):
        for j in range(8):
            ones_v[i, pl.ds(j * 16, 16)] = jnp.ones((16,), jnp.float32)
            czb_v[i, pl.ds(j * 16, 16)] = jnp.zeros((16,), jnp.float32)
        return 0
    lax.fori_loop(0, CHUNK, init_ones, 0)

    for jj in range(5):
        pltpu.sync_copy(czb_v, cnts_sh.at[pl.ds(s * SEG_SLICE + jj * CHUNK, CHUNK)])
    plsc.subcore_barrier()

    bufs = ((ids_v, sem0), (ids_v1, sem1))

    def issue(k, b):
        idv, sem = bufs[b]
        chunk = k * N_WORKERS + wid

        @pl.when(chunk < N_CHUNKS)
        def _():
            pltpu.async_copy(ids_hbm.at[pl.ds(chunk * CHUNK, CHUNK)], idv, sem)

    def process(k, b):
        idv, sem = bufs[b]
        chunk = k * N_WORKERS + wid

        @pl.when(chunk < N_CHUNKS)
        def _():
            pltpu.make_async_copy(ids_hbm.at[pl.ds(0, CHUNK)], idv, sem).wait()
            pltpu.sync_copy(ones_v, cnts_sh.at[idv], add=True)

    issue(0, 0)

    def pair(p, _):
        issue(2 * p + 1, 1)
        process(2 * p, 0)
        issue(2 * p + 2, 0)
        process(2 * p + 1, 1)
        return 0
    lax.fori_loop(0, 40, pair, 0)
    plsc.subcore_barrier()

    pltpu.sync_copy(cnts_sh.at[pl.ds(s * SEG_SLICE, SEG_SLICE)],
                    outc_hbm.at[c, pl.ds(s * SEG_SLICE, SEG_SLICE)])


def _k1_body(x_hbm, ids_hbm, outs_hbm, sums_sh,
             x_v0, x_v1, ids_v0, ids_v1, sem_x0, sem_x1):
    c = lax.axis_index("c")
    s = lax.axis_index("s")
    wid = s * 2 + c

    def zrow(i, _):
        for j in range(8):
            x_v0[i, pl.ds(j * 16, 16)] = jnp.zeros((16,), jnp.float32)
        return 0
    lax.fori_loop(0, CHUNK, zrow, 0)

    for jj in range(5):
        pltpu.sync_copy(x_v0, sums_sh.at[pl.ds(s * SEG_SLICE + jj * CHUNK, CHUNK)])
    plsc.subcore_barrier()

    bufs = ((x_v0, ids_v0, sem_x0), (x_v1, ids_v1, sem_x1))

    def issue(k, b):
        x_v, idv, sem = bufs[b]
        chunk = k * N_WORKERS + wid

        @pl.when(chunk < N_CHUNKS)
        def _():
            base = chunk * CHUNK
            pltpu.sync_copy(ids_hbm.at[pl.ds(base, CHUNK)], idv)
            pltpu.async_copy(x_hbm.at[pl.ds(base, CHUNK)], x_v, sem)

    def process(k, b):
        x_v, idv, sem = bufs[b]
        chunk = k * N_WORKERS + wid

        @pl.when(chunk < N_CHUNKS)
        def _():
            pltpu.make_async_copy(x_hbm.at[pl.ds(0, CHUNK)], x_v, sem).wait()
            pltpu.sync_copy(x_v, sums_sh.at[idv], add=True)

    issue(0, 0)

    def pair(p, _):
        issue(2 * p + 1, 1)
        process(2 * p, 0)
        issue(2 * p + 2, 0)
        process(2 * p + 1, 1)
        return 0
    lax.fori_loop(0, 40, pair, 0)
    plsc.subcore_barrier()

    pltpu.sync_copy(sums_sh.at[pl.ds(s * SEG_SLICE, SEG_SLICE)],
                    outs_hbm.at[c, pl.ds(s * SEG_SLICE, SEG_SLICE)])


def _k2_body(s_ref, c_ref, o_ref):
    tot = s_ref[0] + s_ref[1]
    cnt = c_ref[0, :, 0:1] + c_ref[1, :, 0:1]
    o_ref[...] = -(tot / jnp.maximum(cnt, 1.0))


def _k3_body(x_hbm, ids_hbm, nmean_hbm, out_hbm, mean_sh,
             x_v0, x_v1, x_v2, ids_v0, ids_v1, ids_v2,
             sem_x0, sem_x1, sem_x2, sem_g0, sem_g1, sem_g2,
             sem_w0, sem_w1, sem_w2):
    c = lax.axis_index("c")
    s = lax.axis_index("s")
    wid = s * 2 + c

    for jj, rows in ((0, CHUNK), (1, CHUNK), (2, CHUNK), (3, CHUNK)):
        pltpu.sync_copy(
            nmean_hbm.at[pl.ds(s * SEG_SLICE_K3 + jj * CHUNK, rows)],
            x_v0.at[pl.ds(0, rows)])
        pltpu.sync_copy(
            x_v0.at[pl.ds(0, rows)],
            mean_sh.at[pl.ds(s * SEG_SLICE_K3 + jj * CHUNK, rows)])
    pltpu.sync_copy(nmean_hbm.at[pl.ds(s * SEG_SLICE_K3 + 4 * CHUNK, 120)],
                    x_v0.at[pl.ds(0, 120)])
    pltpu.sync_copy(x_v0.at[pl.ds(0, 120)],
                    mean_sh.at[pl.ds(s * SEG_SLICE_K3 + 4 * CHUNK, 120)])
    plsc.subcore_barrier()

    X = (x_v0, x_v1, x_v2)
    I = (ids_v0, ids_v1, ids_v2)
    SX = (sem_x0, sem_x1, sem_x2)
    SG = (sem_g0, sem_g1, sem_g2)
    SW = (sem_w0, sem_w1, sem_w2)

    def stage(k, b):
        chunk = k * N_WORKERS + wid

        @pl.when(jnp.logical_and(k >= 3, chunk - 3 * N_WORKERS < N_CHUNKS))
        def _():
            pltpu.make_async_copy(X[b], out_hbm.at[pl.ds(0, CHUNK)], SW[b]).wait()

        @pl.when(chunk < N_CHUNKS)
        def _():
            base = chunk * CHUNK
            pltpu.sync_copy(ids_hbm.at[pl.ds(base, CHUNK)], I[b])
            pltpu.async_copy(x_hbm.at[pl.ds(base, CHUNK)], X[b], SX[b])

    def gadd(k, b):
        chunk = k * N_WORKERS + wid

        @pl.when(chunk < N_CHUNKS)
        def _():
            pltpu.make_async_copy(x_hbm.at[pl.ds(0, CHUNK)], X[b], SX[b]).wait()
            pltpu.async_copy(mean_sh.at[I[b]], X[b], SG[b], add=True)

    def write(k, b):
        chunk = k * N_WORKERS + wid

        @pl.when(chunk < N_CHUNKS)
        def _():
            pltpu.make_async_copy(x_hbm.at[pl.ds(0, CHUNK)], X[b], SG[b]).wait()
            pltpu.async_copy(X[b], out_hbm.at[pl.ds(chunk * CHUNK, CHUNK)], SW[b])

    stage(0, 0)
    stage(1, 1)
    gadd(0, 0)

    def tri(t, _):
        s0 = 3 * t + 2
        stage(s0, 2)
        gadd(s0 - 1, 1)
        write(s0 - 2, 0)
        stage(s0 + 1, 0)
        gadd(s0, 2)
        write(s0 - 1, 1)
        stage(s0 + 2, 1)
        gadd(s0 + 1, 0)
        write(s0, 2)
        return 0
    lax.fori_loop(0, 27, tri, 0)


def kernel(x, batch):
    ids = batch.astype(jnp.int32)

    k0 = pl.kernel(
        _k0_body,
        out_type=jax.ShapeDtypeStruct((2, N_SEG_PAD, N_COLS), jnp.float32),
        mesh=_mesh(),
        scratch_types=[
            pltpu.VMEM_SHARED((N_SEG_PAD, N_COLS), jnp.float32),
            pltpu.VMEM((CHUNK,), jnp.int32),
            pltpu.VMEM((CHUNK,), jnp.int32),
            pltpu.VMEM((CHUNK, N_COLS), jnp.float32),
            pltpu.VMEM((CHUNK, N_COLS), jnp.float32),
            pltpu.SemaphoreType.DMA,
            pltpu.SemaphoreType.DMA,
        ],
    )
    part_c = k0(ids)

    k1 = pl.kernel(
        _k1_body,
        out_type=jax.ShapeDtypeStruct((2, N_SEG_PAD, N_COLS), jnp.float32),
        mesh=_mesh(),
        scratch_types=[
            pltpu.VMEM_SHARED((N_SEG_PAD, N_COLS), jnp.float32),
            pltpu.VMEM((CHUNK, N_COLS), jnp.float32),
            pltpu.VMEM((CHUNK, N_COLS), jnp.float32),
            pltpu.VMEM((CHUNK,), jnp.int32),
            pltpu.VMEM((CHUNK,), jnp.int32),
            pltpu.SemaphoreType.DMA,
            pltpu.SemaphoreType.DMA,
        ],
    )
    part_s = k1(x, ids)

    mean = pl.pallas_call(
        _k2_body,
        out_shape=jax.ShapeDtypeStruct((N_SEG_PAD, N_COLS), jnp.float32),
    )(part_s, part_c)

    k3 = pl.kernel(
        _k3_body,
        out_type=jax.ShapeDtypeStruct((N_ROWS, N_COLS), jnp.float32),
        mesh=_mesh(),
        scratch_types=(
            [pltpu.VMEM_SHARED((N_SEG_K3, N_COLS), jnp.float32)]
            + [pltpu.VMEM((CHUNK, N_COLS), jnp.float32)] * 3
            + [pltpu.VMEM((CHUNK,), jnp.int32)] * 3
            + [pltpu.SemaphoreType.DMA] * 9
        ),
    )
    return k3(x, ids, mean)


# speedup vs baseline: 1.9713x; 1.0003x over previous
import jax
import jax.numpy as jnp
from jax import lax
from jax.experimental import pallas as pl
from jax.experimental.pallas import tpu as pltpu
from jax.experimental.pallas import tpu_sc as plsc

N_ROWS = 320000
N_COLS = 128
N_SEG = 10000
N_SEG_PAD = 10240
CHUNK = 128
N_CHUNKS = N_ROWS // CHUNK
N_WORKERS = 32
STEPS = (N_CHUNKS + N_WORKERS - 1) // N_WORKERS
SEG_SLICE = N_SEG_PAD // 16
N_SEG_K3 = 10112
SEG_SLICE_K3 = N_SEG_K3 // 16


def _mesh():
    return plsc.VectorSubcoreMesh(core_axis_name="c", subcore_axis_name="s")


def _k0_body(ids_hbm, outc_hbm, cnts_sh, ids_v, ids_v1, ones_v, czb_v, sem0, sem1):
    c = lax.axis_index("c")
    s = lax.axis_index("s")
    wid = s * 2 + c

    def init_ones(i, _):
        for j in range(8):
            ones_v[i, pl.ds(j * 16, 16)] = jnp.ones((16,), jnp.float32)
            czb_v[i, pl.ds(j * 16, 16)] = jnp.zeros((16,), jnp.float32)
        return 0
    lax.fori_loop(0, CHUNK, init_ones, 0)

    for jj in range(5):
        pltpu.sync_copy(czb_v, cnts_sh.at[pl.ds(s * SEG_SLICE + jj * CHUNK, CHUNK)])
    plsc.subcore_barrier()

    bufs = ((ids_v, sem0), (ids_v1, sem1))

    def issue(k, b):
        idv, sem = bufs[b]
        chunk = k * N_WORKERS + wid

        @pl.when(chunk < N_CHUNKS)
        def _():
            pltpu.async_copy(ids_hbm.at[pl.ds(chunk * CHUNK, CHUNK)], idv, sem)

    def process(k, b):
        idv, sem = bufs[b]
        chunk = k * N_WORKERS + wid

        @pl.when(chunk < N_CHUNKS)
        def _():
            pltpu.make_async_copy(ids_hbm.at[pl.ds(0, CHUNK)], idv, sem).wait()
            pltpu.sync_copy(ones_v, cnts_sh.at[idv], add=True)

    issue(0, 0)

    def pair(p, _):
        issue(2 * p + 1, 1)
        process(2 * p, 0)
        issue(2 * p + 2, 0)
        process(2 * p + 1, 1)
        return 0
    lax.fori_loop(0, 40, pair, 0)
    plsc.subcore_barrier()

    pltpu.sync_copy(cnts_sh.at[pl.ds(s * SEG_SLICE, SEG_SLICE)],
                    outc_hbm.at[c, pl.ds(s * SEG_SLICE, SEG_SLICE)])


def _k1_body(x_hbm, ids_hbm, outs_hbm, sums_sh,
             x_v0, x_v1, ids_v0, ids_v1, sem_x0, sem_x1, sem_s0, sem_s1):
    c = lax.axis_index("c")
    s = lax.axis_index("s")
    wid = s * 2 + c

    def zrow(i, _):
        for j in range(8):
            x_v0[i, pl.ds(j * 16, 16)] = jnp.zeros((16,), jnp.float32)
        return 0
    lax.fori_loop(0, CHUNK, zrow, 0)

    for jj in range(5):
        pltpu.sync_copy(x_v0, sums_sh.at[pl.ds(s * SEG_SLICE + jj * CHUNK, CHUNK)])
    plsc.subcore_barrier()

    bufs = ((x_v0, ids_v0, sem_x0, sem_s0), (x_v1, ids_v1, sem_x1, sem_s1))

    def issue(k, b):
        x_v, idv, sem, sem_s = bufs[b]
        chunk = k * N_WORKERS + wid

        @pl.when(jnp.logical_and(k >= 2, chunk - 2 * N_WORKERS < N_CHUNKS))
        def _():
            pltpu.make_async_copy(x_v, sums_sh.at[idv], sem_s).wait()

        @pl.when(chunk < N_CHUNKS)
        def _():
            base = chunk * CHUNK
            pltpu.sync_copy(ids_hbm.at[pl.ds(base, CHUNK)], idv)
            pltpu.async_copy(x_hbm.at[pl.ds(base, CHUNK)], x_v, sem)

    def process(k, b):
        x_v, idv, sem, sem_s = bufs[b]
        chunk = k * N_WORKERS + wid

        @pl.when(chunk < N_CHUNKS)
        def _():
            pltpu.make_async_copy(x_hbm.at[pl.ds(0, CHUNK)], x_v, sem).wait()
            pltpu.async_copy(x_v, sums_sh.at[idv], sem_s, add=True)

    issue(0, 0)

    def pair(p, _):
        issue(2 * p + 1, 1)
        process(2 * p, 0)
        issue(2 * p + 2, 0)
        process(2 * p + 1, 1)
        return 0
    lax.fori_loop(0, 40, pair, 0)

    @pl.when(79 * N_WORKERS + wid < N_CHUNKS)
    def _():
        pltpu.make_async_copy(x_v1, sums_sh.at[ids_v1], sem_s1).wait()
    plsc.subcore_barrier()

    pltpu.sync_copy(sums_sh.at[pl.ds(s * SEG_SLICE, SEG_SLICE)],
                    outs_hbm.at[c, pl.ds(s * SEG_SLICE, SEG_SLICE)])


def _k2_body(s_ref, c_ref, o_ref):
    tot = s_ref[0] + s_ref[1]
    cnt = c_ref[0, :, 0:1] + c_ref[1, :, 0:1]
    o_ref[...] = -(tot / jnp.maximum(cnt, 1.0))


def _k3_body(x_hbm, ids_hbm, nmean_hbm, out_hbm, mean_sh,
             x_v0, x_v1, x_v2, ids_v0, ids_v1, ids_v2,
             sem_x0, sem_x1, sem_x2, sem_g0, sem_g1, sem_g2,
             sem_w0, sem_w1, sem_w2):
    c = lax.axis_index("c")
    s = lax.axis_index("s")
    wid = s * 2 + c

    for jj, rows in ((0, CHUNK), (1, CHUNK), (2, CHUNK), (3, CHUNK)):
        pltpu.sync_copy(
            nmean_hbm.at[pl.ds(s * SEG_SLICE_K3 + jj * CHUNK, rows)],
            x_v0.at[pl.ds(0, rows)])
        pltpu.sync_copy(
            x_v0.at[pl.ds(0, rows)],
            mean_sh.at[pl.ds(s * SEG_SLICE_K3 + jj * CHUNK, rows)])
    pltpu.sync_copy(nmean_hbm.at[pl.ds(s * SEG_SLICE_K3 + 4 * CHUNK, 120)],
                    x_v0.at[pl.ds(0, 120)])
    pltpu.sync_copy(x_v0.at[pl.ds(0, 120)],
                    mean_sh.at[pl.ds(s * SEG_SLICE_K3 + 4 * CHUNK, 120)])
    plsc.subcore_barrier()

    X = (x_v0, x_v1, x_v2)
    I = (ids_v0, ids_v1, ids_v2)
    SX = (sem_x0, sem_x1, sem_x2)
    SG = (sem_g0, sem_g1, sem_g2)
    SW = (sem_w0, sem_w1, sem_w2)

    def stage(k, b):
        chunk = k * N_WORKERS + wid

        @pl.when(jnp.logical_and(k >= 3, chunk - 3 * N_WORKERS < N_CHUNKS))
        def _():
            pltpu.make_async_copy(X[b], out_hbm.at[pl.ds(0, CHUNK)], SW[b]).wait()

        @pl.when(chunk < N_CHUNKS)
        def _():
            base = chunk * CHUNK
            pltpu.sync_copy(ids_hbm.at[pl.ds(base, CHUNK)], I[b])
            pltpu.async_copy(x_hbm.at[pl.ds(base, CHUNK)], X[b], SX[b])

    def gadd(k, b):
        chunk = k * N_WORKERS + wid

        @pl.when(chunk < N_CHUNKS)
        def _():
            pltpu.make_async_copy(x_hbm.at[pl.ds(0, CHUNK)], X[b], SX[b]).wait()
            pltpu.async_copy(mean_sh.at[I[b]], X[b], SG[b], add=True)

    def write(k, b):
        chunk = k * N_WORKERS + wid

        @pl.when(chunk < N_CHUNKS)
        def _():
            pltpu.make_async_copy(x_hbm.at[pl.ds(0, CHUNK)], X[b], SG[b]).wait()
            pltpu.async_copy(X[b], out_hbm.at[pl.ds(chunk * CHUNK, CHUNK)], SW[b])

    stage(0, 0)
    stage(1, 1)
    gadd(0, 0)

    def tri(t, _):
        s0 = 3 * t + 2
        stage(s0, 2)
        gadd(s0 - 1, 1)
        write(s0 - 2, 0)
        stage(s0 + 1, 0)
        gadd(s0, 2)
        write(s0 - 1, 1)
        stage(s0 + 2, 1)
        gadd(s0 + 1, 0)
        write(s0, 2)
        return 0
    lax.fori_loop(0, 27, tri, 0)


def kernel(x, batch):
    ids = batch.astype(jnp.int32)

    k0 = pl.kernel(
        _k0_body,
        out_type=jax.ShapeDtypeStruct((2, N_SEG_PAD, N_COLS), jnp.float32),
        mesh=_mesh(),
        scratch_types=[
            pltpu.VMEM_SHARED((N_SEG_PAD, N_COLS), jnp.float32),
            pltpu.VMEM((CHUNK,), jnp.int32),
            pltpu.VMEM((CHUNK,), jnp.int32),
            pltpu.VMEM((CHUNK, N_COLS), jnp.float32),
            pltpu.VMEM((CHUNK, N_COLS), jnp.float32),
            pltpu.SemaphoreType.DMA,
            pltpu.SemaphoreType.DMA,
        ],
    )
    part_c = k0(ids)

    k1 = pl.kernel(
        _k1_body,
        out_type=jax.ShapeDtypeStruct((2, N_SEG_PAD, N_COLS), jnp.float32),
        mesh=_mesh(),
        scratch_types=[
            pltpu.VMEM_SHARED((N_SEG_PAD, N_COLS), jnp.float32),
            pltpu.VMEM((CHUNK, N_COLS), jnp.float32),
            pltpu.VMEM((CHUNK, N_COLS), jnp.float32),
            pltpu.VMEM((CHUNK,), jnp.int32),
            pltpu.VMEM((CHUNK,), jnp.int32),
            pltpu.SemaphoreType.DMA,
            pltpu.SemaphoreType.DMA,
            pltpu.SemaphoreType.DMA,
            pltpu.SemaphoreType.DMA,
        ],
    )
    part_s = k1(x, ids)

    mean = pl.pallas_call(
        _k2_body,
        out_shape=jax.ShapeDtypeStruct((N_SEG_PAD, N_COLS), jnp.float32),
    )(part_s, part_c)

    k3 = pl.kernel(
        _k3_body,
        out_type=jax.ShapeDtypeStruct((N_ROWS, N_COLS), jnp.float32),
        mesh=_mesh(),
        scratch_types=(
            [pltpu.VMEM_SHARED((N_SEG_K3, N_COLS), jnp.float32)]
            + [pltpu.VMEM((CHUNK, N_COLS), jnp.float32)] * 3
            + [pltpu.VMEM((CHUNK,), jnp.int32)] * 3
            + [pltpu.SemaphoreType.DMA] * 9
        ),
    )
    return k3(x, ids, mean)


# speedup vs baseline: 2.4002x; 1.2175x over previous
import jax
import jax.numpy as jnp
from jax import lax
from jax.experimental import pallas as pl
from jax.experimental.pallas import tpu as pltpu
from jax.experimental.pallas import tpu_sc as plsc

N_ROWS = 320000
N_COLS = 128
N_SEG = 10000
N_SEG_PAD = 10240
CHUNK = 128
N_CHUNKS = N_ROWS // CHUNK
N_WORKERS = 32
STEPS = (N_CHUNKS + N_WORKERS - 1) // N_WORKERS
SEG_SLICE = N_SEG_PAD // 16
N_SEG_K3 = 10112
SEG_SLICE_K3 = N_SEG_K3 // 16


def _mesh():
    return plsc.VectorSubcoreMesh(core_axis_name="c", subcore_axis_name="s")


def _k1_body(x_hbm, ids_hbm, outs_hbm, oute_hbm, sums_sh,
             x_v0, x_v1, ids_v0, ids_v1, e_v, sem_x0, sem_x1, sem_s0, sem_s1):
    c = lax.axis_index("c")
    s = lax.axis_index("s")
    wid = s * 2 + c

    def zrow(i, _):
        for j in range(8):
            x_v0[i, pl.ds(j * 16, 16)] = jnp.zeros((16,), jnp.float32)
        return 0
    lax.fori_loop(0, CHUNK, zrow, 0)

    def zerow(i, _):
        e_v[pl.ds(i * 16, 16)] = jnp.zeros((16,), jnp.int32)
        return 0
    lax.fori_loop(0, N_SEG_PAD // 16, zerow, 0)

    for jj in range(SEG_SLICE // CHUNK):
        pltpu.sync_copy(x_v0,
                        sums_sh.at[pl.ds(s * SEG_SLICE + jj * CHUNK, CHUNK)])
    plsc.subcore_barrier()

    bufs = ((x_v0, ids_v0, sem_x0, sem_s0), (x_v1, ids_v1, sem_x1, sem_s1))

    def issue(k, b):
        x_v, idv, sem, sem_s = bufs[b]
        ck = k * N_WORKERS + wid

        @pl.when(jnp.logical_and(k >= 2, ck - 2 * N_WORKERS < N_CHUNKS))
        def _():
            pltpu.make_async_copy(x_v, sums_sh.at[idv], sem_s).wait()

        @pl.when(ck < N_CHUNKS)
        def _():
            base = ck * CHUNK
            pltpu.sync_copy(ids_hbm.at[pl.ds(base, CHUNK)], idv)
            pltpu.async_copy(x_hbm.at[pl.ds(base, CHUNK)], x_v, sem)

    def process(k, b):
        x_v, idv, sem, sem_s = bufs[b]
        ck = k * N_WORKERS + wid

        @pl.when(ck < N_CHUNKS)
        def _():
            base = ck * CHUNK
            for j in range(8):
                idvec = idv[pl.ds(j * 16, 16)]
                pos = base + j * 16 + lax.iota(jnp.int32, 16) + 1
                plsc.store_scatter(e_v, [idvec], pos)
            pltpu.make_async_copy(x_hbm.at[pl.ds(0, CHUNK)], x_v, sem).wait()
            pltpu.async_copy(x_v, sums_sh.at[idv], sem_s, add=True)

    issue(0, 0)

    def pair(p, _):
        issue(2 * p + 1, 1)
        process(2 * p, 0)
        issue(2 * p + 2, 0)
        process(2 * p + 1, 1)
        return 0
    lax.fori_loop(0, 39, pair, 0)
    process(78, 0)

    @pl.when(77 * N_WORKERS + wid < N_CHUNKS)
    def _():
        pltpu.make_async_copy(x_v1, sums_sh.at[ids_v1], sem_s1).wait()

    @pl.when(78 * N_WORKERS + wid < N_CHUNKS)
    def _():
        pltpu.make_async_copy(x_v0, sums_sh.at[ids_v0], sem_s0).wait()
    plsc.subcore_barrier()

    pltpu.sync_copy(sums_sh.at[pl.ds(s * SEG_SLICE, SEG_SLICE)],
                    outs_hbm.at[c, pl.ds(s * SEG_SLICE, SEG_SLICE)])
    pltpu.sync_copy(e_v, oute_hbm.at[wid])


def _prefix_max(a, axis, n):
    sh = 1
    while sh < n:
        kept = lax.slice_in_dim(a, 0, n - sh, axis=axis)
        z_shape = list(a.shape)
        z_shape[axis] = sh
        shifted = jnp.concatenate(
            [jnp.zeros(z_shape, a.dtype), kept], axis=axis)
        a = jnp.maximum(a, shifted)
        sh *= 2
    return a


def _k2a_body(e_ref, c_ref):
    m = jnp.max(e_ref[...], axis=0)
    wp = _prefix_max(m, 1, 128)
    rowmax = wp[:, 127:128]
    rp = _prefix_max(rowmax, 0, 80)
    carry = jnp.concatenate(
        [jnp.zeros((1, 1), jnp.int32), rp[:-1]], axis=0)
    inc = jnp.maximum(wp, carry)
    prev = jnp.concatenate([carry, inc[:, :-1]], axis=1)
    cnt = inc - prev
    c_ref[...] = jnp.maximum(cnt, 1).astype(jnp.float32)


def _k2_body(s_ref, c_ref, o_ref):
    tot = s_ref[0] + s_ref[1]
    o_ref[...] = -(tot / c_ref[...])


def _k3_body(x_hbm, ids_hbm, nmean_hbm, out_hbm, mean_sh,
             x_v0, x_v1, x_v2, ids_v0, ids_v1, ids_v2,
             sem_x0, sem_x1, sem_x2, sem_g0, sem_g1, sem_g2,
             sem_w0, sem_w1, sem_w2):
    c = lax.axis_index("c")
    s = lax.axis_index("s")
    wid = s * 2 + c

    for jj, rows in ((0, CHUNK), (1, CHUNK), (2, CHUNK), (3, CHUNK)):
        pltpu.sync_copy(
            nmean_hbm.at[pl.ds(s * SEG_SLICE_K3 + jj * CHUNK, rows)],
            x_v0.at[pl.ds(0, rows)])
        pltpu.sync_copy(
            x_v0.at[pl.ds(0, rows)],
            mean_sh.at[pl.ds(s * SEG_SLICE_K3 + jj * CHUNK, rows)])
    pltpu.sync_copy(nmean_hbm.at[pl.ds(s * SEG_SLICE_K3 + 4 * CHUNK, 120)],
                    x_v0.at[pl.ds(0, 120)])
    pltpu.sync_copy(x_v0.at[pl.ds(0, 120)],
                    mean_sh.at[pl.ds(s * SEG_SLICE_K3 + 4 * CHUNK, 120)])
    plsc.subcore_barrier()

    X = (x_v0, x_v1, x_v2)
    I = (ids_v0, ids_v1, ids_v2)
    SX = (sem_x0, sem_x1, sem_x2)
    SG = (sem_g0, sem_g1, sem_g2)
    SW = (sem_w0, sem_w1, sem_w2)

    def stage(k, b):
        chunk = k * N_WORKERS + wid

        @pl.when(jnp.logical_and(k >= 3, chunk - 3 * N_WORKERS < N_CHUNKS))
        def _():
            pltpu.make_async_copy(X[b], out_hbm.at[pl.ds(0, CHUNK)], SW[b]).wait()

        @pl.when(chunk < N_CHUNKS)
        def _():
            base = chunk * CHUNK
            pltpu.sync_copy(ids_hbm.at[pl.ds(base, CHUNK)], I[b])
            pltpu.async_copy(x_hbm.at[pl.ds(base, CHUNK)], X[b], SX[b])

    def gadd(k, b):
        chunk = k * N_WORKERS + wid

        @pl.when(chunk < N_CHUNKS)
        def _():
            pltpu.make_async_copy(x_hbm.at[pl.ds(0, CHUNK)], X[b], SX[b]).wait()
            pltpu.async_copy(mean_sh.at[I[b]], X[b], SG[b], add=True)

    def write(k, b):
        chunk = k * N_WORKERS + wid

        @pl.when(chunk < N_CHUNKS)
        def _():
            pltpu.make_async_copy(x_hbm.at[pl.ds(0, CHUNK)], X[b], SG[b]).wait()
            pltpu.async_copy(X[b], out_hbm.at[pl.ds(chunk * CHUNK, CHUNK)], SW[b])

    stage(0, 0)
    stage(1, 1)
    gadd(0, 0)

    def tri(t, _):
        s0 = 3 * t + 2
        stage(s0, 2)
        gadd(s0 - 1, 1)
        write(s0 - 2, 0)
        stage(s0 + 1, 0)
        gadd(s0, 2)
        write(s0 - 1, 1)
        stage(s0 + 2, 1)
        gadd(s0 + 1, 0)
        write(s0, 2)
        return 0
    lax.fori_loop(0, 27, tri, 0)


def kernel(x, batch):
    ids = batch.astype(jnp.int32)

    k1 = pl.kernel(
        _k1_body,
        out_type=(
            jax.ShapeDtypeStruct((2, N_SEG_PAD, N_COLS), jnp.float32),
            jax.ShapeDtypeStruct((N_WORKERS, N_SEG_PAD), jnp.int32),
        ),
        mesh=_mesh(),
        compiler_params=pltpu.CompilerParams(needs_layout_passes=False),
        scratch_types=[
            pltpu.VMEM_SHARED((N_SEG_PAD, N_COLS), jnp.float32),
            pltpu.VMEM((CHUNK, N_COLS), jnp.float32),
            pltpu.VMEM((CHUNK, N_COLS), jnp.float32),
            pltpu.VMEM((CHUNK,), jnp.int32),
            pltpu.VMEM((CHUNK,), jnp.int32),
            pltpu.VMEM((N_SEG_PAD,), jnp.int32),
            pltpu.SemaphoreType.DMA,
            pltpu.SemaphoreType.DMA,
            pltpu.SemaphoreType.DMA,
            pltpu.SemaphoreType.DMA,
        ],
    )
    part_s, ends = k1(x, ids)

    cnt = pl.pallas_call(
        _k2a_body,
        out_shape=jax.ShapeDtypeStruct((N_SEG_PAD // 128, 128), jnp.float32),
    )(ends.reshape(N_WORKERS, N_SEG_PAD // 128, 128))

    mean = pl.pallas_call(
        _k2_body,
        out_shape=jax.ShapeDtypeStruct((N_SEG_PAD, N_COLS), jnp.float32),
    )(part_s, cnt.reshape(N_SEG_PAD, 1))

    k3 = pl.kernel(
        _k3_body,
        out_type=jax.ShapeDtypeStruct((N_ROWS, N_COLS), jnp.float32),
        mesh=_mesh(),
        scratch_types=(
            [pltpu.VMEM_SHARED((N_SEG_K3, N_COLS), jnp.float32)]
            + [pltpu.VMEM((CHUNK, N_COLS), jnp.float32)] * 3
            + [pltpu.VMEM((CHUNK,), jnp.int32)] * 3
            + [pltpu.SemaphoreType.DMA] * 9
        ),
    )
    return k3(x, ids, mean)
